# baseline (speedup 1.0000x reference)
import functools

import jax
import jax.numpy as jnp
import numpy as np
from jax import lax
from jax.experimental import pallas as pl
from jax.experimental.pallas import tpu as pltpu
from jax.experimental.pallas import tpu_sc as plsc

N = 32768
NT = 16
CH = N // NT
VR = CH // 16
RB = 5
B = 1 << RB
NPASS = 7
MSB = np.int32(-2147483648)


def _iota16():
    return lax.iota(jnp.int32, 16)


def _sc_body(k_hbm, out_hbm, kv, pv, kg, posv, histv, offv, totv, totmine,
             zv, bw, flagmine, flagv, okv, sk0, sp0, sk1, sp1, stot, sflag):
    t = lax.axis_index("s")
    it = _iota16()
    base = t * CH
    zero16 = jnp.zeros((16,), jnp.int32)
    ones16 = jnp.ones((16,), jnp.int32)

    pltpu.sync_copy(k_hbm.at[pl.ds(base, CH)], kv)

    def init_j(j, _):
        pv[pl.ds(j * 16, 16)] = base + j * 16 + it
        zv[pl.ds(j * 16, 16)] = zero16
        return 0

    lax.fori_loop(0, VR, init_j, 0)

    bufs = [(sk0, sp0), (sk1, sp1)]
    for p in range(NPASS):
        shift = RB * p
        k_in, p_in = bufs[p % 2]
        k_out, p_out = bufs[(p + 1) % 2]
        if p > 0:
            pltpu.sync_copy(k_in.at[pl.ds(base, CH)], kv)
            pltpu.sync_copy(p_in.at[pl.ds(base, CH)], pv)
        pltpu.sync_copy(zv, k_out.at[pl.ds(base, CH)])
        pltpu.sync_copy(zv, p_out.at[pl.ds(base, CH)])
        for b in range(B):
            histv[b * 16:(b + 1) * 16] = zero16

        def hist_j(j, _):
            kvec = plsc.load_gather(kv, [it * VR + j])
            d = lax.shift_right_logical(kvec, shift) & (B - 1)
            plsc.addupdate_scatter(histv, [d * 16 + it], ones16)
            return 0

        lax.fori_loop(0, VR, hist_j, 0)

        tv0 = zero16
        tv1 = zero16
        for d in range(B):
            s_d = jnp.sum(histv[d * 16:(d + 1) * 16])
            sel = jnp.where(it == (d % 16), s_d, 0)
            if d < 16:
                tv0 = tv0 + sel
            else:
                tv1 = tv1 + sel
        totmine[0:16] = tv0
        totmine[16:32] = tv1
        pltpu.sync_copy(totmine, stot.at[pl.ds(t * B, B)])
        plsc.subcore_barrier()

        pltpu.sync_copy(stot, totv)

        def scan_d(d, basec):
            col = plsc.load_gather(totv, [it * B + d])
            colcs = plsc.cumsum(col)
            s_tot = jnp.sum(col)
            prev_t = jnp.sum(jnp.where(it == t, colcs - col, 0))
            h_d = histv[pl.ds(d * 16, 16)]
            lane_excl = plsc.cumsum(h_d) - h_d
            offv[pl.ds(d * 16, 16)] = lane_excl + basec + prev_t
            return basec + s_tot

        lax.fori_loop(0, B, scan_d, jnp.int32(0))

        def perm_j(j, _):
            kvec = plsc.load_gather(kv, [it * VR + j])
            d = lax.shift_right_logical(kvec, shift) & (B - 1)
            cidx = d * 16 + it
            pos = plsc.load_gather(offv, [cidx])
            plsc.store_scatter(offv, [cidx], pos + 1)
            plsc.store_scatter(posv, [it, zero16 + j], pos)
            return 0

        lax.fori_loop(0, VR, perm_j, 0)

        for i in range(NT):
            pltpu.sync_copy(kv.at[pl.ds(i * VR, VR)],
                            k_out.at[posv.at[i]], add=True)
            pltpu.sync_copy(pv.at[pl.ds(i * VR, VR)],
                            p_out.at[posv.at[i]], add=True)
        plsc.subcore_barrier()

    ks, ps = bufs[NPASS % 2]
    pltpu.sync_copy(ks.at[pl.ds(base, CH)], kv)
    pltpu.sync_copy(ps.at[pl.ds(base, CH)], pv)
    for i in range(NT):
        pltpu.sync_copy(k_hbm.at[pv.at[pl.ds(i * VR, VR)]],
                        kg.at[pl.ds(i * VR, VR)])

    def chk_j(j, bad):
        gk = kg[pl.ds(j * 16, 16)]
        kk = kv[pl.ds(j * 16, 16)]
        bad = bad + jnp.sum(jnp.where(gk == kk, 0, 1))
        nxt = plsc.load_gather(kv, [jnp.minimum(j * 16 + it + 1, CH - 1)])
        bad = bad + jnp.sum(jnp.where((kk ^ MSB) <= (nxt ^ MSB), 0, 1))
        return bad

    bad = lax.fori_loop(0, VR, chk_j, jnp.int32(0))

    @pl.when(t < NT - 1)
    def _():
        pltpu.sync_copy(ks.at[pl.ds((t + 1) * CH, 16)], bw)

    lastv = plsc.load_gather(kv, [zero16 + (CH - 1)])
    bvec = bw[0:16]
    viol = jnp.where((it == 0) & ((lastv ^ MSB) > (bvec ^ MSB)), 1, 0)
    bad = bad + jnp.sum(viol) * jnp.where(t < NT - 1, 1, 0)

    flagmine[0:16] = jnp.where(it == 0, bad, 0)
    pltpu.sync_copy(flagmine, sflag.at[pl.ds(t * 16, 16)])
    plsc.subcore_barrier()

    @pl.when(t == 0)
    def _():
        pltpu.sync_copy(sflag, flagv)

        def red_i(i, acc):
            return acc + jnp.sum(flagv[pl.ds(i * 16, 16)])

        tot_bad = lax.fori_loop(0, NT, red_i, jnp.int32(0))
        okv[0:16] = jnp.where(zero16 + tot_bad == 0, 1, 0)
        pltpu.sync_copy(okv, out_hbm)


_sc_sort = functools.partial(
    pl.kernel,
    out_type=jax.ShapeDtypeStruct((16,), jnp.int32),
    mesh=plsc.VectorSubcoreMesh(
        core_axis_name="c", subcore_axis_name="s", num_cores=1),
    compiler_params=pltpu.CompilerParams(needs_layout_passes=False),
    scratch_types=[
        pltpu.VMEM((CH,), jnp.int32),
        pltpu.VMEM((CH,), jnp.int32),
        pltpu.VMEM((CH,), jnp.int32),
        pltpu.VMEM((NT, VR), jnp.int32),
        pltpu.VMEM((B * 16,), jnp.int32),
        pltpu.VMEM((B * 16,), jnp.int32),
        pltpu.VMEM((NT * B,), jnp.int32),
        pltpu.VMEM((B,), jnp.int32),
        pltpu.VMEM((CH,), jnp.int32),
        pltpu.VMEM((16,), jnp.int32),
        pltpu.VMEM((16,), jnp.int32),
        pltpu.VMEM((NT * 16,), jnp.int32),
        pltpu.VMEM((16,), jnp.int32),
        pltpu.VMEM_SHARED((N,), jnp.int32),
        pltpu.VMEM_SHARED((N,), jnp.int32),
        pltpu.VMEM_SHARED((N,), jnp.int32),
        pltpu.VMEM_SHARED((N,), jnp.int32),
        pltpu.VMEM_SHARED((NT * B,), jnp.int32),
        pltpu.VMEM_SHARED((NT * 16,), jnp.int32),
    ],
)(_sc_body)


def kernel(x):
    u = lax.bitcast_convert_type(x, jnp.int32)
    s = lax.shift_right_arithmetic(u, 31)
    keys = (u ^ ~s) & (s | jnp.int32(0x7FFFFFFF))
    out = _sc_sort(keys)
    return out[0].astype(jnp.bool_)


# speedup vs baseline: 2.2225x; 2.2225x over previous
import functools

import jax
import jax.numpy as jnp
import numpy as np
from jax import lax
from jax.experimental import pallas as pl
from jax.experimental.pallas import tpu as pltpu
from jax.experimental.pallas import tpu_sc as plsc

N = 32768
NT = 16
CH = N // NT
VR = CH // 16
RB = 8
B = 1 << RB
NG = B // 16
NPASS = 4
MSB = np.int32(-2147483648)


def _iota16():
    return lax.iota(jnp.int32, 16)


def _sc_body(k_hbm, iota_hbm, out_hbm, kv, pv, kg, posv, histv, offv, totv,
             bw, flagmine, flagv, okv, sem, semp,
             sk0, sp0, sk1, sp1, korig, stot, sflag):
    t = lax.axis_index("s")
    it = _iota16()
    base = t * CH
    zero16 = jnp.zeros((16,), jnp.int32)

    pv_in = pltpu.async_copy(iota_hbm.at[pl.ds(base, CH)], pv, semp)
    pltpu.sync_copy(k_hbm.at[pl.ds(base, CH)], kv)
    pltpu.sync_copy(kv, korig.at[pl.ds(base, CH)])

    bufs = [(sk0, sp0), (sk1, sp1)]
    for p in range(NPASS):
        shift = RB * p
        k_out, p_out = bufs[p % 2]
        if p > 0:
            k_in, p_in = bufs[(p + 1) % 2]
            pv_in = pltpu.async_copy(p_in.at[pl.ds(base, CH)], pv, semp)
            pltpu.sync_copy(k_in.at[pl.ds(base, CH)], kv)

        for g in range(NG):
            histv[pl.ds(g * 16, 16)] = zero16

        def hist_j(j, _):
            kvec = kv[pl.ds(j * 16, 16)]
            d = lax.shift_right_logical(kvec, shift) & (B - 1)
            occ, last = plsc.scan_count(d)
            plsc.addupdate_scatter(histv, [d], occ, mask=last)
            return 0

        lax.fori_loop(0, VR, hist_j, 0)

        pltpu.sync_copy(histv, stot.at[pl.ds(t * B, B)])
        plsc.subcore_barrier()
        pltpu.sync_copy(stot, totv)

        def scan_g(g, carry):
            tot_g = zero16
            prev_g = zero16
            for tp in range(NT):
                row = totv[pl.ds(tp * B + g * 16, 16)]
                tot_g = tot_g + row
                prev_g = prev_g + jnp.where(
                    lax.full((16,), tp, jnp.int32) < t, row, 0)
            base_g = plsc.cumsum(tot_g) - tot_g + carry
            offv[pl.ds(g * 16, 16)] = base_g + prev_g
            return carry + jnp.sum(tot_g)

        lax.fori_loop(0, NG, scan_g, jnp.int32(0))

        def perm_j(j, _):
            kvec = kv[pl.ds(j * 16, 16)]
            d = lax.shift_right_logical(kvec, shift) & (B - 1)
            occ, last = plsc.scan_count(d)
            pos = plsc.load_gather(offv, [d]) + occ - 1
            jd = lax.div(j, 8)
            jm = j - jd * 8
            plsc.store_scatter(posv, [zero16 + jd, jm * 16 + it], pos)
            plsc.addupdate_scatter(offv, [d], occ, mask=last)
            return 0

        lax.fori_loop(0, VR, perm_j, 0)

        pv_in.wait()
        copies = []
        for i in range(NT):
            copies.append(pltpu.async_copy(
                kv.at[pl.ds(i * VR, VR)], k_out.at[posv.at[i]], sem))
            copies.append(pltpu.async_copy(
                pv.at[pl.ds(i * VR, VR)], p_out.at[posv.at[i]], sem))
        for c in copies:
            c.wait()
        plsc.subcore_barrier()

    ks, ps = bufs[(NPASS - 1) % 2]
    pltpu.sync_copy(ks.at[pl.ds(base, CH)], kv)
    pltpu.sync_copy(ps.at[pl.ds(base, CH)], pv)
    copies = []
    for i in range(NT):
        copies.append(pltpu.async_copy(
            korig.at[pv.at[pl.ds(i * VR, VR)]], kg.at[pl.ds(i * VR, VR)],
            sem))
    for c in copies:
        c.wait()

    def chk_j(j, bad):
        gk = kg[pl.ds(j * 16, 16)]
        kk = kv[pl.ds(j * 16, 16)]
        bad = bad + jnp.sum(jnp.where(gk == kk, 0, 1))
        nxt = plsc.load_gather(kv, [jnp.minimum(j * 16 + it + 1, CH - 1)])
        bad = bad + jnp.sum(jnp.where((kk ^ MSB) <= (nxt ^ MSB), 0, 1))
        return bad

    bad = lax.fori_loop(0, VR, chk_j, jnp.int32(0))

    @pl.when(t < NT - 1)
    def _():
        pltpu.sync_copy(ks.at[pl.ds((t + 1) * CH, 16)], bw)

    lastv = plsc.load_gather(kv, [zero16 + (CH - 1)])
    bvec = bw[0:16]
    viol = jnp.where((it == 0) & ((lastv ^ MSB) > (bvec ^ MSB)), 1, 0)
    bad = bad + jnp.sum(viol) * jnp.where(t < NT - 1, 1, 0)

    flagmine[0:16] = jnp.where(it == 0, bad, 0)
    pltpu.sync_copy(flagmine, sflag.at[pl.ds(t * 16, 16)])
    plsc.subcore_barrier()

    @pl.when(t == 0)
    def _():
        pltpu.sync_copy(sflag, flagv)

        def red_i(i, acc):
            return acc + jnp.sum(flagv[pl.ds(i * 16, 16)])

        tot_bad = lax.fori_loop(0, NT, red_i, jnp.int32(0))
        okv[0:16] = jnp.where(zero16 + tot_bad == 0, 1, 0)
        pltpu.sync_copy(okv, out_hbm)


_sc_sort = functools.partial(
    pl.kernel,
    out_type=jax.ShapeDtypeStruct((16,), jnp.int32),
    mesh=plsc.VectorSubcoreMesh(
        core_axis_name="c", subcore_axis_name="s", num_cores=1),
    compiler_params=pltpu.CompilerParams(needs_layout_passes=False),
    scratch_types=[
        pltpu.VMEM((CH,), jnp.int32),
        pltpu.VMEM((CH,), jnp.int32),
        pltpu.VMEM((CH,), jnp.int32),
        pltpu.VMEM((NT, VR), jnp.int32),
        pltpu.VMEM((B,), jnp.int32),
        pltpu.VMEM((B,), jnp.int32),
        pltpu.VMEM((NT * B,), jnp.int32),
        pltpu.VMEM((16,), jnp.int32),
        pltpu.VMEM((16,), jnp.int32),
        pltpu.VMEM((NT * 16,), jnp.int32),
        pltpu.VMEM((16,), jnp.int32),
        pltpu.SemaphoreType.DMA,
        pltpu.SemaphoreType.DMA,
        pltpu.VMEM_SHARED((N,), jnp.int32),
        pltpu.VMEM_SHARED((N,), jnp.int32),
        pltpu.VMEM_SHARED((N,), jnp.int32),
        pltpu.VMEM_SHARED((N,), jnp.int32),
        pltpu.VMEM_SHARED((N,), jnp.int32),
        pltpu.VMEM_SHARED((NT * B,), jnp.int32),
        pltpu.VMEM_SHARED((NT * 16,), jnp.int32),
    ],
)(_sc_body)


def kernel(x):
    u = lax.bitcast_convert_type(x, jnp.int32)
    s = lax.shift_right_arithmetic(u, 31)
    keys = (u ^ ~s) & (s | jnp.int32(0x7FFFFFFF))
    iota = lax.iota(jnp.int32, N)
    out = _sc_sort(keys, iota)
    return out[0].astype(jnp.bool_)


# speedup vs baseline: 2.3152x; 1.0417x over previous
import functools

import jax
import jax.numpy as jnp
import numpy as np
from jax import lax
from jax.experimental import pallas as pl
from jax.experimental.pallas import tpu as pltpu
from jax.experimental.pallas import tpu_sc as plsc

N = 32768
NT = 16
CH = N // NT
VR = CH // 16
RB = 8
B = 1 << RB
NG = B // 16
NPASS = 4
MSB = np.int32(-2147483648)


def _iota16():
    return lax.iota(jnp.int32, 16)


def _sc_body(u_hbm, out_hbm, kv, pv, kg, posv, dv, occv, lv, histv, offv,
             totv, bw, flagmine, flagv, okv, sem, semp, semk,
             sk0, sp0, sk1, sp1, korig, stot, sflag):
    t = lax.axis_index("s")
    it = _iota16()
    base = t * CH
    zero16 = jnp.zeros((16,), jnp.int32)

    pltpu.sync_copy(u_hbm.at[pl.ds(base, CH)], kv)

    def key_j(j, _):
        u = kv[pl.ds(j * 16, 16)]
        s = lax.shift_right_arithmetic(u, 31)
        kv[pl.ds(j * 16, 16)] = (u ^ ~s) & (s | jnp.int32(0x7FFFFFFF))
        pv[pl.ds(j * 16, 16)] = base + j * 16 + it
        return 0

    lax.fori_loop(0, VR, key_j, 0)
    korig_in = pltpu.async_copy(kv, korig.at[pl.ds(base, CH)], semk)

    bufs = [(sk0, sp0), (sk1, sp1)]
    pv_in = None
    for p in range(NPASS):
        shift = RB * p
        k_out, p_out = bufs[p % 2]
        if p > 0:
            k_in, p_in = bufs[(p + 1) % 2]
            pv_in = pltpu.async_copy(p_in.at[pl.ds(base, CH)], pv, semp)
            pltpu.sync_copy(k_in.at[pl.ds(base, CH)], kv)

        for g in range(NG):
            histv[pl.ds(g * 16, 16)] = zero16

        def hist_j(j, _):
            kvec = kv[pl.ds(j * 16, 16)]
            d = lax.shift_right_logical(kvec, shift) & (B - 1)
            occ, last = plsc.scan_count(d)
            dv[pl.ds(j * 16, 16)] = d
            occv[pl.ds(j * 16, 16)] = occ
            lv[pl.ds(j * 16, 16)] = jnp.where(last, 1, 0)
            plsc.addupdate_scatter(histv, [d], occ, mask=last)
            return 0

        lax.fori_loop(0, VR, hist_j, 0)

        pltpu.sync_copy(histv, stot.at[pl.ds(t * B, B)])
        plsc.subcore_barrier()
        pltpu.sync_copy(stot, totv)

        def scan_g(g, carry):
            tot_g = zero16
            prev_g = zero16
            for tp in range(NT):
                row = totv[pl.ds(tp * B + g * 16, 16)]
                tot_g = tot_g + row
                prev_g = prev_g + jnp.where(
                    lax.full((16,), tp, jnp.int32) < t, row, 0)
            base_g = plsc.cumsum(tot_g) - tot_g + carry
            offv[pl.ds(g * 16, 16)] = base_g + prev_g
            return carry + jnp.sum(tot_g)

        lax.fori_loop(0, NG, scan_g, jnp.int32(0))

        def perm_j(j, _):
            d = dv[pl.ds(j * 16, 16)]
            occ = occv[pl.ds(j * 16, 16)]
            last = lv[pl.ds(j * 16, 16)] != 0
            pos = plsc.load_gather(offv, [d]) + occ - 1
            jd = lax.div(j, 8)
            jm = j - jd * 8
            plsc.store_scatter(posv, [zero16 + jd, jm * 16 + it], pos)
            plsc.addupdate_scatter(offv, [d], occ, mask=last)
            return 0

        lax.fori_loop(0, VR, perm_j, 0)

        if pv_in is not None:
            pv_in.wait()
        copies = []
        for i in range(NT):
            copies.append(pltpu.async_copy(
                kv.at[pl.ds(i * VR, VR)], k_out.at[posv.at[i]], sem))
            copies.append(pltpu.async_copy(
                pv.at[pl.ds(i * VR, VR)], p_out.at[posv.at[i]], sem))
        for c in copies:
            c.wait()
        plsc.subcore_barrier()

    ks, ps = bufs[(NPASS - 1) % 2]
    pltpu.sync_copy(ks.at[pl.ds(base, CH)], kv)
    pltpu.sync_copy(ps.at[pl.ds(base, CH)], pv)
    korig_in.wait()
    copies = []
    for i in range(NT):
        copies.append(pltpu.async_copy(
            korig.at[pv.at[pl.ds(i * VR, VR)]], kg.at[pl.ds(i * VR, VR)],
            sem))

    def order_j(j, bad):
        kk = kv[pl.ds(j * 16, 16)]
        nxt = plsc.load_gather(kv, [jnp.minimum(j * 16 + it + 1, CH - 1)])
        return bad + jnp.sum(jnp.where((kk ^ MSB) <= (nxt ^ MSB), 0, 1))

    bad = lax.fori_loop(0, VR, order_j, jnp.int32(0))
    for c in copies:
        c.wait()

    def chk_j(j, bad):
        gk = kg[pl.ds(j * 16, 16)]
        kk = kv[pl.ds(j * 16, 16)]
        return bad + jnp.sum(jnp.where(gk == kk, 0, 1))

    bad = lax.fori_loop(0, VR, chk_j, bad)

    @pl.when(t < NT - 1)
    def _():
        pltpu.sync_copy(ks.at[pl.ds((t + 1) * CH, 16)], bw)

    lastv = plsc.load_gather(kv, [zero16 + (CH - 1)])
    bvec = bw[0:16]
    viol = jnp.where((it == 0) & ((lastv ^ MSB) > (bvec ^ MSB)), 1, 0)
    bad = bad + jnp.sum(viol) * jnp.where(t < NT - 1, 1, 0)

    flagmine[0:16] = jnp.where(it == 0, bad, 0)
    pltpu.sync_copy(flagmine, sflag.at[pl.ds(t * 16, 16)])
    plsc.subcore_barrier()

    @pl.when(t == 0)
    def _():
        pltpu.sync_copy(sflag, flagv)

        def red_i(i, acc):
            return acc + jnp.sum(flagv[pl.ds(i * 16, 16)])

        tot_bad = lax.fori_loop(0, NT, red_i, jnp.int32(0))
        okv[0:16] = jnp.where(zero16 + tot_bad == 0, 1, 0)
        pltpu.sync_copy(okv, out_hbm)


_sc_sort = functools.partial(
    pl.kernel,
    out_type=jax.ShapeDtypeStruct((16,), jnp.int32),
    mesh=plsc.VectorSubcoreMesh(
        core_axis_name="c", subcore_axis_name="s", num_cores=1),
    compiler_params=pltpu.CompilerParams(needs_layout_passes=False),
    scratch_types=[
        pltpu.VMEM((CH,), jnp.int32),
        pltpu.VMEM((CH,), jnp.int32),
        pltpu.VMEM((CH,), jnp.int32),
        pltpu.VMEM((NT, VR), jnp.int32),
        pltpu.VMEM((CH,), jnp.int32),
        pltpu.VMEM((CH,), jnp.int32),
        pltpu.VMEM((CH,), jnp.int32),
        pltpu.VMEM((B,), jnp.int32),
        pltpu.VMEM((B,), jnp.int32),
        pltpu.VMEM((NT * B,), jnp.int32),
        pltpu.VMEM((16,), jnp.int32),
        pltpu.VMEM((16,), jnp.int32),
        pltpu.VMEM((NT * 16,), jnp.int32),
        pltpu.VMEM((16,), jnp.int32),
        pltpu.SemaphoreType.DMA,
        pltpu.SemaphoreType.DMA,
        pltpu.SemaphoreType.DMA,
        pltpu.VMEM_SHARED((N,), jnp.int32),
        pltpu.VMEM_SHARED((N,), jnp.int32),
        pltpu.VMEM_SHARED((N,), jnp.int32),
        pltpu.VMEM_SHARED((N,), jnp.int32),
        pltpu.VMEM_SHARED((N,), jnp.int32),
        pltpu.VMEM_SHARED((NT * B,), jnp.int32),
        pltpu.VMEM_SHARED((NT * 16,), jnp.int32),
    ],
)(_sc_body)


def kernel(x):
    out = _sc_sort(lax.bitcast_convert_type(x, jnp.int32))
    return out[0].astype(jnp.bool_)
